# baseline (speedup 1.0000x reference)
import functools

import jax
import jax.numpy as jnp
from jax import lax
from jax.experimental import pallas as pl
from jax.experimental.pallas import tpu as pltpu
from jax.experimental.pallas import tpu_sc as plsc

_L = 16
_NEG = -1e30


def _softmax_into(tbl_ref, nvec, valid):
    lanes = lax.iota(jnp.int32, _L)
    v = []
    for i in range(nvec):
        x = tbl_ref[pl.ds(_L * i, _L)]
        nvalid = valid - _L * i
        if nvalid < _L:
            x = jnp.where(lanes < nvalid, x, _NEG)
        v.append(x)
    m = v[0]
    for x in v[1:]:
        m = jnp.maximum(m, x)
    mmax = jnp.max(m)
    e = [jnp.exp(x - mmax) for x in v]
    t = e[0]
    for x in e[1:]:
        t = t + x
    inv = 1.0 / jnp.broadcast_to(jnp.sum(t), (_L,))
    for i in range(nvec):
        tbl_ref[pl.ds(_L * i, _L)] = e[i] * inv


def _run(nn_t, ln_flat, lognp, loglp):
    rows, b = nn_t.shape
    l_tot = ln_flat.shape[0]
    n_tbl = lognp.shape[0]
    l_tbl = loglp.shape[0]
    info = plsc.get_sparse_core_info()
    nw = info.num_cores * info.num_subcores
    bpw = b // nw
    lpw = l_tot // nw
    mesh = plsc.VectorSubcoreMesh(core_axis_name="c", subcore_axis_name="s")

    @functools.partial(
        pl.kernel,
        mesh=mesh,
        out_type=[
            jax.ShapeDtypeStruct((rows, b), jnp.float32),
            jax.ShapeDtypeStruct((l_tot,), jnp.float32),
        ],
        scratch_types=[
            pltpu.VMEM((rows, bpw), jnp.int32),
            pltpu.VMEM((lpw,), jnp.int32),
            pltpu.VMEM((rows, bpw), jnp.float32),
            pltpu.VMEM((lpw,), jnp.float32),
            pltpu.VMEM((64,), jnp.float32),
            pltpu.VMEM((16,), jnp.float32),
            pltpu.SemaphoreType.DMA,
            pltpu.SemaphoreType.DMA,
            pltpu.SemaphoreType.DMA,
            pltpu.SemaphoreType.DMA,
        ],
        compiler_params=pltpu.CompilerParams(needs_layout_passes=False),
    )
    def sc_kernel(nn_hbm, ln_hbm, lognp_hbm, loglp_hbm, out_n_hbm, out_l_hbm,
                  nidx, lidx, nout, lout, ntbl, ltbl,
                  sem_n, sem_l, sem_t, sem_u):
        wid = lax.axis_index("s") * info.num_cores + lax.axis_index("c")
        base = wid * bpw
        cp_n = pltpu.async_copy(nn_hbm.at[:, pl.ds(base, bpw)], nidx, sem_n)
        cp_l = pltpu.async_copy(ln_hbm.at[pl.ds(base, lpw)], lidx, sem_l)
        cp_t = pltpu.async_copy(lognp_hbm, ntbl.at[pl.ds(0, n_tbl)], sem_t)
        cp_u = pltpu.async_copy(loglp_hbm, ltbl.at[pl.ds(0, l_tbl)], sem_u)
        cp_t.wait()
        cp_u.wait()
        _softmax_into(ntbl, 4, n_tbl)
        _softmax_into(ltbl, 1, l_tbl)
        cp_n.wait()
        for i in range(bpw // _L):
            off = i * _L
            for r in range(rows):
                idx = nidx[r, pl.ds(off, _L)] - 1
                nout[r, pl.ds(off, _L)] = plsc.load_gather(ntbl, [idx])
        co_n = pltpu.async_copy(nout, out_n_hbm.at[:, pl.ds(base, bpw)], sem_t)
        cp_l.wait()
        for i in range(lpw // _L):
            off = i * _L
            idx = lidx[pl.ds(off, _L)] - 1
            lout[pl.ds(off, _L)] = plsc.load_gather(ltbl, [idx])
        co_l = pltpu.async_copy(lout, out_l_hbm.at[pl.ds(base, lpw)], sem_u)
        co_n.wait()
        co_l.wait()

    return sc_kernel(nn_t, ln_flat, lognp, loglp)


def kernel(normal_numbers, lucky_number, log_normal_probs, log_lucky_probs):
    out_t, out_l = _run(normal_numbers.T, lucky_number.reshape(-1),
                        log_normal_probs, log_lucky_probs)
    return (out_t.T, out_l.reshape(lucky_number.shape))


# speedup vs baseline: 1.0512x; 1.0512x over previous
import functools

import jax
import jax.numpy as jnp
from jax import lax
from jax.experimental import pallas as pl
from jax.experimental.pallas import tpu as pltpu
from jax.experimental.pallas import tpu_sc as plsc

_L = 16
_NEG = -1e30


def _softmax_into(tbl_ref, nvec, valid):
    lanes = lax.iota(jnp.int32, _L)
    v = []
    for i in range(nvec):
        x = tbl_ref[pl.ds(_L * i, _L)]
        nvalid = valid - _L * i
        if nvalid < _L:
            x = jnp.where(lanes < nvalid, x, _NEG)
        v.append(x)
    m = v[0]
    for x in v[1:]:
        m = jnp.maximum(m, x)
    mmax = jnp.max(m)
    e = [jnp.exp(x - mmax) for x in v]
    t = e[0]
    for x in e[1:]:
        t = t + x
    inv = 1.0 / jnp.broadcast_to(jnp.sum(t), (_L,))
    for i in range(nvec):
        tbl_ref[pl.ds(_L * i, _L)] = e[i] * inv


def _run(nn_t, ln_flat, lognp, loglp):
    rows, b = nn_t.shape
    l_tot = ln_flat.shape[0]
    n_tbl = lognp.shape[0]
    l_tbl = loglp.shape[0]
    info = plsc.get_sparse_core_info()
    nw = info.num_cores * info.num_subcores
    bpw = b // nw
    lpw = l_tot // nw
    mesh = plsc.VectorSubcoreMesh(core_axis_name="c", subcore_axis_name="s")

    @functools.partial(
        pl.kernel,
        mesh=mesh,
        out_type=[
            jax.ShapeDtypeStruct((rows, b), jnp.float32),
            jax.ShapeDtypeStruct((l_tot,), jnp.float32),
        ],
        scratch_types=[
            pltpu.VMEM((rows, bpw), jnp.int32),
            pltpu.VMEM((lpw,), jnp.int32),
            pltpu.VMEM((rows, bpw), jnp.float32),
            pltpu.VMEM((lpw,), jnp.float32),
            pltpu.VMEM((64,), jnp.float32),
            pltpu.VMEM((16,), jnp.float32),
            pltpu.SemaphoreType.DMA,
            pltpu.SemaphoreType.DMA,
            pltpu.SemaphoreType.DMA,
            pltpu.SemaphoreType.DMA,
        ],
        compiler_params=pltpu.CompilerParams(needs_layout_passes=False),
    )
    def sc_kernel(nn_hbm, ln_hbm, lognp_hbm, loglp_hbm, out_n_hbm, out_l_hbm,
                  nidx, lidx, nout, lout, ntbl, ltbl,
                  sem_n, sem_l, sem_t, sem_u):
        wid = lax.axis_index("s") * info.num_cores + lax.axis_index("c")
        base = wid * bpw
        cp_n = pltpu.async_copy(nn_hbm.at[:, pl.ds(base, bpw)], nidx, sem_n)
        cp_l = pltpu.async_copy(ln_hbm.at[pl.ds(base, lpw)], lidx, sem_l)
        cp_t = pltpu.async_copy(lognp_hbm, ntbl.at[pl.ds(0, n_tbl)], sem_t)
        cp_u = pltpu.async_copy(loglp_hbm, ltbl.at[pl.ds(0, l_tbl)], sem_u)
        cp_t.wait()
        cp_u.wait()
        _softmax_into(ntbl, 4, n_tbl)
        _softmax_into(ltbl, 1, l_tbl)
        cp_n.wait()

        def nbody(i, carry):
            off = pl.multiple_of(i * _L, _L)
            for r in range(rows):
                idx = nidx[r, pl.ds(off, _L)] - 1
                nout[r, pl.ds(off, _L)] = plsc.load_gather(ntbl, [idx])
            return carry

        lax.fori_loop(0, bpw // _L, nbody, 0, unroll=2)
        co_n = pltpu.async_copy(nout, out_n_hbm.at[:, pl.ds(base, bpw)], sem_t)
        cp_l.wait()

        def lbody(i, carry):
            off = pl.multiple_of(i * _L, _L)
            idx = lidx[pl.ds(off, _L)] - 1
            lout[pl.ds(off, _L)] = plsc.load_gather(ltbl, [idx])
            return carry

        lax.fori_loop(0, lpw // _L, lbody, 0, unroll=4)
        co_l = pltpu.async_copy(lout, out_l_hbm.at[pl.ds(base, lpw)], sem_u)
        co_n.wait()
        co_l.wait()

    return sc_kernel(nn_t, ln_flat, lognp, loglp)


def kernel(normal_numbers, lucky_number, log_normal_probs, log_lucky_probs):
    out_t, out_l = _run(normal_numbers.T, lucky_number.reshape(-1),
                        log_normal_probs, log_lucky_probs)
    return (out_t.T, out_l.reshape(lucky_number.shape))
